# baseline (speedup 1.0000x reference)
import functools

import jax
import jax.numpy as jnp
from jax import lax
from jax.experimental import pallas as pl
from jax.experimental.pallas import tpu as pltpu
from jax.experimental.pallas import tpu_sc as plsc

N = 10000
D = 128
DH = D // 2
E = 500000
NC, NS, L = 2, 16, 16
NW = NC * NS
CHUNK = 128
NCH = -(-E // (NS * CHUNK))
EPAD = NS * NCH * CHUNK
NCH2 = -(-E // (NW * CHUNK))
EPAD2 = NW * NCH2 * CHUNK
NROWS = N
STRIPE = NROWS // NS
SSIZES = (128, 128, 128, 128, 113)
NPAD = EPAD - E
NB = 3
NBODY = (NCH // NB) * NB

_f32 = jnp.float32
_i32 = jnp.int32


def _segsum_body(src_h, dst_h, tbl_h, sum_h,
                 src_v, dst_v, rows_v, acc_sh, gsems, ssems):
    cid = lax.axis_index("c")
    sid = lax.axis_index("s")

    zero16 = jnp.zeros((L,), _f32)

    @pl.loop(0, CHUNK)
    def _(i):
        for j in range(DH // L):
            rows_v[0, i, pl.ds(j * L, L)] = zero16

    off = 0
    for sz in SSIZES:
        sl = pl.ds(sid * STRIPE + off, sz)
        pltpu.sync_copy(rows_v.at[0, pl.ds(0, sz)], acc_sh.at[sl])
        off += sz
    plsc.subcore_barrier()

    pltpu.sync_copy(src_h.at[sid], src_v)
    pltpu.sync_copy(dst_h.at[sid], dst_v)

    for b in range(NB):
        pltpu.async_copy(tbl_h.at[cid].at[src_v.at[b]], rows_v.at[b],
                         gsems[b])

    @pl.loop(0, NBODY // NB)
    def _(r):
        for b in range(NB):
            jj = r * NB + b
            pltpu.make_async_copy(tbl_h.at[cid].at[src_v.at[jj]],
                                  rows_v.at[b], gsems[b]).wait()
            d1 = pltpu.async_copy(rows_v.at[b], acc_sh.at[dst_v.at[jj]],
                                  ssems[b], add=True)
            d1.wait()

            @pl.when(jj + NB < NCH)
            def _():
                pltpu.async_copy(tbl_h.at[cid].at[src_v.at[jj + NB]],
                                 rows_v.at[b], gsems[b])

    for jj in range(NBODY, NCH):
        b = jj % NB
        pltpu.make_async_copy(tbl_h.at[cid].at[src_v.at[jj]],
                              rows_v.at[b], gsems[b]).wait()
        pltpu.sync_copy(rows_v.at[b], acc_sh.at[dst_v.at[jj]], add=True)

    plsc.subcore_barrier()

    off = 0
    for sz in SSIZES:
        sl = pl.ds(sid * STRIPE + off, sz)
        pltpu.sync_copy(acc_sh.at[sl], sum_h.at[cid, sl])
        off += sz


@functools.cache
def _get_segsum():
    return pl.kernel(
        _segsum_body,
        out_type=jax.ShapeDtypeStruct((NC, NROWS, DH), _f32),
        mesh=plsc.VectorSubcoreMesh(core_axis_name="c", subcore_axis_name="s",
                                    num_cores=NC, num_subcores=NS),
        compiler_params=pltpu.CompilerParams(use_tc_tiling_on_sc=False),
        scratch_types=[
            pltpu.VMEM((NCH, CHUNK), _i32),
            pltpu.VMEM((NCH, CHUNK), _i32),
            pltpu.VMEM((NB, CHUNK, DH), _f32),
            pltpu.VMEM_SHARED((NROWS, DH), _f32),
            tuple(pltpu.SemaphoreType.DMA for _ in range(NB)),
            tuple(pltpu.SemaphoreType.DMA for _ in range(NB)),
        ],
    )


CROWS = 640
CSTRIPE = CROWS // NS
NB2 = 3


def _counts_body(row_h, lane_h, eye_h, cnt_h,
                 row_v, lane_v, buf_v, zrow_v, cnt_sh, gsems, ssems):
    cid = lax.axis_index("c")
    sid = lax.axis_index("s")
    wid = cid * NS + sid

    zero16 = jnp.zeros((L,), _f32)

    @pl.loop(0, CSTRIPE)
    def _(i):
        zrow_v[i, :] = zero16

    pltpu.sync_copy(zrow_v, cnt_sh.at[pl.ds(sid * CSTRIPE, CSTRIPE)])
    plsc.subcore_barrier()

    pltpu.sync_copy(row_h.at[wid], row_v)
    pltpu.sync_copy(lane_h.at[wid], lane_v)

    for b in range(NB2):
        pltpu.async_copy(eye_h.at[lane_v.at[b]], buf_v.at[b], gsems[b])

    @pl.loop(0, NCH2 // NB2)
    def _(r):
        for b in range(NB2):
            jj = r * NB2 + b
            pltpu.make_async_copy(eye_h.at[lane_v.at[jj]], buf_v.at[b],
                                  gsems[b]).wait()
            d1 = pltpu.async_copy(buf_v.at[b], cnt_sh.at[row_v.at[jj]],
                                  ssems[b], add=True)
            d1.wait()

            @pl.when(jj + NB2 < NCH2)
            def _():
                pltpu.async_copy(eye_h.at[lane_v.at[jj + NB2]], buf_v.at[b],
                                 gsems[b])

    plsc.subcore_barrier()

    pltpu.sync_copy(cnt_sh.at[pl.ds(sid * CSTRIPE, CSTRIPE)],
                    cnt_h.at[cid, pl.ds(sid * CSTRIPE, CSTRIPE)])


@functools.cache
def _get_counts():
    return pl.kernel(
        _counts_body,
        out_type=jax.ShapeDtypeStruct((NC, CROWS, L), _f32),
        mesh=plsc.VectorSubcoreMesh(core_axis_name="c", subcore_axis_name="s",
                                    num_cores=NC, num_subcores=NS),
        compiler_params=pltpu.CompilerParams(use_tc_tiling_on_sc=False),
        scratch_types=[
            pltpu.VMEM((NCH2, CHUNK), _i32),
            pltpu.VMEM((NCH2, CHUNK), _i32),
            pltpu.VMEM((NB2, CHUNK, L), _f32),
            pltpu.VMEM((CSTRIPE, L), _f32),
            pltpu.VMEM_SHARED((CROWS, L), _f32),
            tuple(pltpu.SemaphoreType.DMA for _ in range(NB2)),
            tuple(pltpu.SemaphoreType.DMA for _ in range(NB2)),
        ],
    )


_BM = 1000
_GRID = N // _BM


def _dotT(a, b):
    return lax.dot_general(a, b, (((1,), (1,)), ((), ())),
                           preferred_element_type=_f32)


def _full(shape):
    return pl.BlockSpec(shape, lambda i: (0,) * len(shape))


def _rows(shape):
    if len(shape) == 3:
        return pl.BlockSpec(shape, lambda i: (0, i, 0))
    return pl.BlockSpec(shape, lambda i: (i, 0))


def _split(p_ref, x):
    p_ref[0] = x[:, :DH]
    p_ref[1] = x[:, DH:]


def _inv_cnt(c_ref):
    return 1.0 / jnp.maximum(c_ref[0] + c_ref[1], 1.0)


def _seg_corr(s_ref, t_ref):
    s = jnp.concatenate([s_ref[0], s_ref[1]], axis=1)
    t = jnp.concatenate([t_ref[0], t_ref[1]], axis=1)
    row = pl.program_id(0) * _BM + lax.broadcasted_iota(_i32, (_BM, 1), 0)
    return s - jnp.where(row < NPAD, 1.0, 0.0) * t


def _mm0_body(x_ref, w_ref, o_ref):
    _split(o_ref, _dotT(x_ref[...], w_ref[...]))


_mm0 = pl.pallas_call(
    _mm0_body,
    grid=(_GRID,),
    in_specs=[_rows((_BM, D)), _full((D, D))],
    out_specs=_rows((NC, _BM, DH)),
    out_shape=jax.ShapeDtypeStruct((NC, N, DH), _f32),
)


def _st1_body(s_ref, t_ref, c_ref, ue_ref, w1r_ref, b1l_ref, w2l_ref,
              ux_ref, p2_ref):
    s = _seg_corr(s_ref, t_ref)
    r1 = _dotT(ue_ref[...], w1r_ref[...])
    ux = jnp.maximum(s * _inv_cnt(c_ref) + b1l_ref[...] + r1, 0.0)
    ux_ref[...] = ux
    _split(p2_ref, _dotT(ux, w2l_ref[...]))


_st1 = pl.pallas_call(
    _st1_body,
    grid=(_GRID,),
    in_specs=[_rows((NC, _BM, DH)), _rows((NC, _BM, DH)), _rows((NC, _BM, 1)),
              _full((1, D)), _full((D, D)), _full((1, D)), _full((D, D))],
    out_specs=[_rows((_BM, D)), _rows((NC, _BM, DH))],
    out_shape=[jax.ShapeDtypeStruct((N, D), _f32),
               jax.ShapeDtypeStruct((NC, N, DH), _f32)],
)


def _st2_body(s_ref, t_ref, c_ref, xm_ref, w2r_ref, b2l_ref, w3l_ref,
              wl2_ref, bl2_ref, p3_ref, zm_ref):
    s = _seg_corr(s_ref, t_ref)
    mx = jnp.maximum(
        s * _inv_cnt(c_ref) + b2l_ref[...] + _dotT(xm_ref[...], w2r_ref[...]),
        0.0)
    _split(p3_ref, _dotT(mx, w3l_ref[...]))
    zm_ref[...] = _dotT(mx, wl2_ref[...]) + bl2_ref[...]


_st2 = pl.pallas_call(
    _st2_body,
    grid=(_GRID,),
    in_specs=[_rows((NC, _BM, DH)), _rows((NC, _BM, DH)), _rows((NC, _BM, 1)),
              _rows((_BM, D)),
              _full((D, D)), _full((1, D)), _full((D, D)),
              _full((64, D)), _full((1, 64))],
    out_specs=[_rows((NC, _BM, DH)), _rows((_BM, 64))],
    out_shape=[jax.ShapeDtypeStruct((NC, N, DH), _f32),
               jax.ShapeDtypeStruct((N, 64), _f32)],
)


def _st3_body(s_ref, t_ref, c_ref, ux_ref, w3r_ref, b3l_ref, wl1_ref,
              bl1_ref, zu_ref):
    s = _seg_corr(s_ref, t_ref)
    ux2 = jnp.maximum(
        s * _inv_cnt(c_ref) + b3l_ref[...] + _dotT(ux_ref[...], w3r_ref[...]),
        0.0)
    zu_ref[...] = _dotT(ux2, wl1_ref[...]) + bl1_ref[...]


_st3 = pl.pallas_call(
    _st3_body,
    grid=(_GRID,),
    in_specs=[_rows((NC, _BM, DH)), _rows((NC, _BM, DH)), _rows((NC, _BM, 1)),
              _rows((_BM, D)),
              _full((D, D)), _full((1, D)), _full((64, D)), _full((1, 64))],
    out_specs=_rows((_BM, 64)),
    out_shape=jax.ShapeDtypeStruct((N, 64), _f32),
)


def _pad_edges(src, dst):
    ar = jnp.arange(NPAD, dtype=_i32)
    src3 = jnp.concatenate([src.astype(_i32), ar])
    dst3 = jnp.concatenate([dst.astype(_i32), ar])
    return src3.reshape(NS, NCH, CHUNK), dst3.reshape(NS, NCH, CHUNK)


def _pad_dst2(dst):
    pad = EPAD2 - E
    dst2 = jnp.concatenate([dst.astype(_i32), jnp.full((pad,), N, _i32)])
    dst2 = dst2.reshape(NW, NCH2, CHUNK)
    return dst2 >> 4, dst2 & 15


def kernel(x_movie, x_user, edge_rates, edge_rev, tuples_coo, user_emb,
           W1l, b1l, W1r, W2l, b2l, W2r, W3l, b3l, W3r,
           Wlin1, blin1, Wlin2, blin2):
    n_users = x_user.shape[0]
    srcR, dstR = _pad_edges(edge_rev[0], edge_rev[1])
    srcA, dstA = _pad_edges(edge_rates[0], edge_rates[1])

    ue = user_emb.reshape(1, D)
    b1 = b1l.reshape(1, D)
    b2 = b2l.reshape(1, D)
    b3 = b3l.reshape(1, D)
    bz1 = blin1.reshape(1, 64)
    bz2 = blin2.reshape(1, 64)

    eye = jnp.eye(L, dtype=_f32)
    rR, lR = _pad_dst2(edge_rev[1])
    rA, lA = _pad_dst2(edge_rates[1])
    c1 = _get_counts()(rR, lR, eye).reshape(NC, CROWS * L, 1)
    c2 = _get_counts()(rA, lA, eye).reshape(NC, CROWS * L, 1)

    p1 = _mm0(x_movie, W1l)
    s1 = _get_segsum()(srcR, dstR, p1)
    ux, p2 = _st1(s1, p1, c1, ue, W1r, b1, W2l)

    s2 = _get_segsum()(srcA, dstA, p2)
    p3, zm = _st2(s2, p2, c2, x_movie, W2r, b2, W3l, Wlin2, bz2)

    s3 = _get_segsum()(srcR, dstR, p3)
    zu = _st3(s3, p3, c1, ux, W3r, b3, Wlin1, bz1)

    X = jnp.concatenate([zu, zm], axis=0)
    new_index = jnp.vstack((tuples_coo[0], tuples_coo[1] + n_users))
    return (X, new_index)
